# baseline (speedup 1.0000x reference)
import jax
import jax.numpy as jnp
from jax.experimental import pallas as pl

K = 32


def _neg_dist_kernel(q_ref, s_ref, q2_ref, s2_ref, out_ref):
    BQ = q_ref.shape[1]
    qi = pl.program_id(1)
    q = q_ref[0]
    s = s_ref[0]
    dot = jax.lax.dot_general(q, s, (((1,), (1,)), ((), ())),
                              preferred_element_type=jnp.float32)
    q2 = q2_ref[0, 0, pl.ds(qi * BQ, BQ)]
    d2 = (s2_ref[0, 0][None, :] + q2[:, None]) - 2.0 * dot
    out_ref[0] = -jnp.sqrt(jnp.maximum(d2, 0.0))


def _neg_dist(query_xyz, support_xyz):
    B, Nq, _ = query_xyz.shape
    Ns = support_xyz.shape[1]
    BQ = 256
    qp = jnp.pad(query_xyz, ((0, 0), (0, 0), (0, 5)))
    sp = jnp.pad(support_xyz, ((0, 0), (0, 0), (0, 5)))
    q2 = jnp.sum(query_xyz ** 2, axis=-1).reshape(B, 1, Nq)
    s2 = jnp.sum(support_xyz ** 2, axis=-1).reshape(B, 1, Ns)
    return pl.pallas_call(
        _neg_dist_kernel,
        grid=(B, Nq // BQ),
        in_specs=[
            pl.BlockSpec((1, BQ, 8), lambda b, q: (b, q, 0)),
            pl.BlockSpec((1, Ns, 8), lambda b, q: (b, 0, 0)),
            pl.BlockSpec((1, 1, Nq), lambda b, q: (b, 0, 0)),
            pl.BlockSpec((1, 1, Ns), lambda b, q: (b, 0, 0)),
        ],
        out_specs=pl.BlockSpec((1, BQ, Ns), lambda b, q: (b, q, 0)),
        out_shape=jax.ShapeDtypeStruct((B, Nq, Ns), jnp.float32),
    )(qp, sp, q2, s2)


def _group(feats, idx):
    B, C, N = feats.shape
    M, Kk = idx.shape[1], idx.shape[2]
    idxb = jnp.broadcast_to(idx[:, None, :, :], (B, C, M, Kk)).reshape(B, C, M * Kk)
    return jnp.take_along_axis(feats, idxb, axis=2).reshape(B, C, M, Kk)


def kernel(query_xyz, support_xyz, features):
    neg = _neg_dist(query_xyz, support_xyz)
    _, idx = jax.lax.top_k(neg, K)
    idx = idx.astype(jnp.int32)
    xyz_trans = jnp.transpose(support_xyz, (0, 2, 1))
    grouped_xyz = _group(xyz_trans, idx)
    grouped_xyz = grouped_xyz - jnp.transpose(query_xyz, (0, 2, 1))[:, :, :, None]
    grouped_features = _group(features, idx)
    return (grouped_xyz, grouped_features)


# speedup vs baseline: 301.1026x; 301.1026x over previous
import functools

import jax
import jax.numpy as jnp
from jax import lax
from jax.experimental import pallas as pl
from jax.experimental.pallas import tpu as pltpu
from jax.experimental.pallas import tpu_sc as plsc

K = 32


def _topk_kernel(q_ref, s_ref, q2_ref, s2_ref, out_ref):
    BQ = q_ref.shape[1]
    Ns = s_ref.shape[1]
    G = Ns // 128
    qi = pl.program_id(1)
    q = q_ref[0]
    s = s_ref[0]
    dot = jax.lax.dot_general(q, s, (((1,), (1,)), ((), ())),
                              preferred_element_type=jnp.float32)
    q2 = q2_ref[0, 0, pl.ds(qi * BQ, BQ)]
    d2 = (s2_ref[0, 0][None, :] + q2[:, None]) - 2.0 * dot
    dist = jnp.sqrt(jnp.maximum(d2, 0.0))
    key = jax.lax.bitcast_convert_type(dist, jnp.int32).reshape(BQ, G, 128)
    gio = jax.lax.broadcasted_iota(jnp.int32, (BQ, G, 128), 1)
    lio = jax.lax.broadcasted_iota(jnp.int32, (BQ, G, 128), 2)
    iota = gio * 128 + lio
    BIG = jnp.int32(0x7FFFFFFF)
    for k in range(K):
        m1 = jnp.min(key, axis=1)
        m = jnp.min(m1, axis=1)
        cand = jnp.where(key == m[:, None, None], iota, BIG)
        c1 = jnp.min(cand, axis=1)
        idx_k = jnp.min(c1, axis=1)
        out_ref[0, :, pl.ds(k, 1)] = idx_k[:, None]
        key = jnp.where(iota == idx_k[:, None, None], BIG, key)


def _knn_idx(query_xyz, support_xyz):
    B, Nq, _ = query_xyz.shape
    Ns = support_xyz.shape[1]
    BQ = 256
    qp = jnp.pad(query_xyz, ((0, 0), (0, 0), (0, 5)))
    sp = jnp.pad(support_xyz, ((0, 0), (0, 0), (0, 5)))
    q2 = jnp.sum(query_xyz ** 2, axis=-1).reshape(B, 1, Nq)
    s2 = jnp.sum(support_xyz ** 2, axis=-1).reshape(B, 1, Ns)
    return pl.pallas_call(
        _topk_kernel,
        grid=(B, Nq // BQ),
        in_specs=[
            pl.BlockSpec((1, BQ, 8), lambda b, q: (b, q, 0)),
            pl.BlockSpec((1, Ns, 8), lambda b, q: (b, 0, 0)),
            pl.BlockSpec((1, 1, Nq), lambda b, q: (b, 0, 0)),
            pl.BlockSpec((1, 1, Ns), lambda b, q: (b, 0, 0)),
        ],
        out_specs=pl.BlockSpec((1, BQ, K), lambda b, q: (b, q, 0)),
        out_shape=jax.ShapeDtypeStruct((B, Nq, K), jnp.int32),
    )(qp, sp, q2, s2)


_NC, _NS = 2, 16
_QCH = 8
_RB = 4


def _sc_gather(idx2, feats, xyzt):
    B, C, Ns = feats.shape
    E = idx2.shape[1]
    CH = E // _QCH
    idx_flat = idx2.reshape(B * E)
    feats_flat = feats.reshape(B * C * Ns)
    xyz_flat = xyzt.reshape(B * 3 * Ns)
    mesh = plsc.VectorSubcoreMesh(core_axis_name="c", subcore_axis_name="s")

    @functools.partial(
        pl.kernel,
        mesh=mesh,
        out_type=(
            jax.ShapeDtypeStruct((B * C * E,), jnp.float32),
            jax.ShapeDtypeStruct((B * 3 * E,), jnp.float32),
        ),
        scratch_types=[
            pltpu.VMEM((CH,), jnp.int32),
            pltpu.VMEM((_RB * Ns,), jnp.float32),
            pltpu.VMEM((_RB * CH,), jnp.float32),
        ],
        compiler_params=pltpu.CompilerParams(needs_layout_passes=False),
    )
    def k(idx_hbm, feats_hbm, xyz_hbm, gf_hbm, gx_hbm, idx_v, rows_v, out_v):
        wid = lax.axis_index("s") * _NC + lax.axis_index("c")
        b = wid // _QCH
        base = (wid % _QCH) * CH
        pltpu.sync_copy(idx_hbm.at[pl.ds(wid * CH, CH)], idx_v)

        def gather_rows(nrows):
            def body(i, _):
                iv = idx_v[pl.ds(i * 16, 16)]
                for r in range(nrows):
                    vals = plsc.load_gather(rows_v, [iv + jnp.int32(r * Ns)])
                    out_v[pl.ds(r * CH + i * 16, 16)] = vals
                return 0
            lax.fori_loop(0, CH // 16, body, 0)

        def feat_block(g, _):
            c0 = g * _RB
            pltpu.sync_copy(
                feats_hbm.at[pl.ds((b * C + c0) * Ns, _RB * Ns)], rows_v)
            gather_rows(_RB)
            for r in range(_RB):
                pltpu.sync_copy(
                    out_v.at[pl.ds(r * CH, CH)],
                    gf_hbm.at[pl.ds((b * C + c0 + r) * E + base, CH)])
            return 0

        lax.fori_loop(0, C // _RB, feat_block, 0)

        pltpu.sync_copy(xyz_hbm.at[pl.ds(b * 3 * Ns, 3 * Ns)],
                        rows_v.at[pl.ds(0, 3 * Ns)])
        gather_rows(3)
        for r in range(3):
            pltpu.sync_copy(
                out_v.at[pl.ds(r * CH, CH)],
                gx_hbm.at[pl.ds((b * 3 + r) * E + base, CH)])

    gf, gx = k(idx_flat, feats_flat, xyz_flat)
    return gf.reshape(B, C, E), gx.reshape(B, 3, E)


def kernel(query_xyz, support_xyz, features):
    B, Nq, _ = query_xyz.shape
    C = features.shape[1]
    idx = _knn_idx(query_xyz, support_xyz)
    xyz_trans = jnp.transpose(support_xyz, (0, 2, 1))
    gf, gx = _sc_gather(idx.reshape(B, Nq * K), features, xyz_trans)
    grouped_xyz = gx.reshape(B, 3, Nq, K)
    grouped_xyz = grouped_xyz - jnp.transpose(query_xyz, (0, 2, 1))[:, :, :, None]
    grouped_features = gf.reshape(B, C, Nq, K)
    return (grouped_xyz, grouped_features)
